# baseline (speedup 1.0000x reference)
import functools

import jax
import jax.numpy as jnp
from jax import lax
from jax.experimental import pallas as pl
from jax.experimental.pallas import tpu as pltpu
from jax.experimental.pallas import tpu_sc as plsc

N = 10000
E = 320000
D = 128
NPAD = 10240
NR = NPAD // 128
NC = 2
NS = 16
EA = E // NS
CA = 2000
EW = E // (NC * NS)
CB = 80
NCH = EW // CB
BCH = 25
NB = NCH // BCH

_i32 = jnp.int32
_f32 = jnp.float32


def _rowcol(idx):
  return [lax.shift_right_logical(idx, 7), lax.bitwise_and(idx, 127)]


def _bfs_body(e0_hbm, e1_hbm, init_hbm, zero_hbm,
              enc_out,
              cur_v, all_v, new_v, tmp_v, e0c_v, e1c_v, init_v,
              stage_v, idvec_v, sh_hits):
  t = lax.axis_index("s")
  ones = jnp.ones((16,), _i32)
  lanes = jnp.arange(16, dtype=_i32)

  pltpu.sync_copy(zero_hbm, cur_v)
  pltpu.sync_copy(zero_hbm, all_v)
  pltpu.sync_copy(init_hbm, init_v)
  iv = init_v[...]
  plsc.store_scatter(cur_v, _rowcol(iv), ones)
  plsc.store_scatter(all_v, _rowcol(iv), ones)

  for g in range(NR // 16):
    idvec_v[pl.ds(16 * g, 16)] = lanes + 16 * g

  pltpu.sync_copy(e0_hbm.at[pl.ds(t * EA, EA)], e0c_v)
  pltpu.sync_copy(e1_hbm.at[pl.ds(t * EA, EA)], e1c_v)

  for _hop in range(2):
    pltpu.sync_copy(zero_hbm, new_v)

    @pl.when(t < 10)
    def _zero_hits():
      pltpu.sync_copy(zero_hbm.at[pl.ds(8 * t, 8)], sh_hits.at[pl.ds(8 * t, 8)])

    plsc.subcore_barrier()

    def chunk_body(ci, carry):
      for g in range(5):
        gi = 80 * ci + 16 * g
        a = e0c_v[pl.ds(gi, 16)]
        b = e1c_v[pl.ds(gi, 16)]
        sm = plsc.load_gather(cur_v, _rowcol(a))
        dm = plsc.load_gather(cur_v, _rowcol(b))
        plsc.store_scatter(new_v, _rowcol(b), ones, mask=sm > 0)
        plsc.store_scatter(new_v, _rowcol(a), ones, mask=dm > 0)
      return carry

    lax.fori_loop(0, EA // 80, chunk_body, 0)
    pl.delay(200)
    pltpu.sync_copy(new_v, sh_hits.at[idvec_v], add=True)
    plsc.subcore_barrier()
    pltpu.sync_copy(sh_hits, tmp_v)

    def rb_body(i, carry):
      for j in range(8):
        h = tmp_v[i, pl.ds(16 * j, 16)]
        c = (h > 0).astype(_i32)
        cur_v[i, pl.ds(16 * j, 16)] = c
        all_v[i, pl.ds(16 * j, 16)] = lax.bitwise_or(all_v[i, pl.ds(16 * j, 16)], c)
      return carry

    lax.fori_loop(0, NR, rb_body, 0)
    plsc.subcore_barrier()

  def pre_body(i, acc):
    for j in range(8):
      acc = acc + all_v[i, pl.ds(16 * j, 16)]
    return acc

  acc0 = lax.fori_loop(0, 8 * t, pre_body, jnp.zeros((16,), _i32))
  run0 = jnp.full((16,), jnp.sum(acc0), _i32)

  def cs_body(i_loc, run):
    for j in range(8):
      x = all_v[8 * t + i_loc, pl.ds(16 * j, 16)]
      pre = plsc.cumsum(x) + run
      stage_v[i_loc, pl.ds(16 * j, 16)] = lax.bitwise_or(
          lax.shift_left(pre, 1), x)
      run = run + jnp.full((16,), jnp.sum(x), _i32)
    return run

  nrows = jnp.where(t < 10, 8, 0)
  lax.fori_loop(0, nrows, cs_body, run0)
  pl.delay(200)

  @pl.when(t < 10)
  def _writeback():
    pltpu.sync_copy(stage_v, enc_out.at[pl.ds(8 * t, 8)])


def _msg_body(node_hbm, e0r_hbm, e1r_hbm, mr_hbm, enc_hbm, zf_hbm,
              pk_out, part_out,
              enc_v, e0t_v, e1t_v, mt_v, sbuf_a, sbuf_b,
              rows_a, rows_b, sh_agg, semg_a, semg_b, sems_a, sems_b):
  cid = lax.axis_index("c")
  t = lax.axis_index("s")
  wid = cid * NS + t
  neg1 = jnp.full((16,), -1, _i32)
  ones = jnp.ones((16,), _i32)
  lanes = jnp.arange(16, dtype=_i32)
  zf = jnp.zeros((16,), _f32)

  pltpu.sync_copy(enc_hbm, enc_v)

  @pl.when(t < 10)
  def _zero_agg():
    pltpu.sync_copy(zf_hbm, sh_agg.at[pl.ds(1000 * t, 1000)])

  plsc.subcore_barrier()

  def issue_gather(ci, rows_v, sem):
    return pltpu.async_copy(node_hbm.at[e0t_v.at[ci]], rows_v, sem)

  def wait_gather(ci, rows_v, sem):
    pltpu.make_async_copy(node_hbm.at[e0t_v.at[ci]], rows_v, sem).wait()

  def compute_scale(ci, rows_v):
    keeps = []
    for g in range(CB // 16):
      a = e0t_v[ci, pl.ds(16 * g, 16)]
      b = e1t_v[ci, pl.ds(16 * g, 16)]
      m = plsc.bitcast(mt_v[ci, pl.ds(16 * g, 16)], _f32)
      enc0 = plsc.load_gather(enc_v, _rowcol(a))
      enc1 = plsc.load_gather(enc_v, _rowcol(b))
      keep = lax.bitwise_and(lax.bitwise_and(enc0, enc1), ones)
      keeps.append((enc0, enc1, keep))
      w = keep.astype(_f32) / (1.0 + jnp.exp(-m))

      @plsc.parallel_loop(0, 16, 1, unroll=2)
      def _scale(r):
        ws = jnp.take_along_axis(w, jnp.full((16,), r, _i32), axis=0)
        for k in range(8):
          rows_v[16 * g + r, pl.ds(16 * k, 16)] = (
              rows_v[16 * g + r, pl.ds(16 * k, 16)] * ws)

    return keeps

  def emit_outputs(ci, keeps, sbuf_v):
    kbit = jnp.full((16,), 1 << 28, _i32)
    zi = jnp.zeros((16,), _i32)
    for g in range(CB // 16):
      enc0, enc1, keep = keeps[g]
      b = e1t_v[ci, pl.ds(16 * g, 16)]
      kb = keep > 0
      sbuf_v[pl.ds(16 * g, 16)] = jnp.where(kb, b, neg1)
      hi = lax.bitwise_or(
          lax.bitwise_or(lax.shift_right_logical(enc0, 1),
                         lax.shift_left(lax.shift_right_logical(enc1, 1), 14)),
          kbit)
      mt_v[ci, pl.ds(16 * g, 16)] = jnp.where(kb, hi, zi)

  def issue_scatter(rows_v, sbuf_v, sem):
    pl.delay(100)
    dst = sh_agg.at[plsc.Indices(sbuf_v, ignored_value=-1)]
    return pltpu.async_copy(rows_v, dst, sem, add=True)

  def wait_scatter(rows_v, sbuf_v, sem):
    dst = sh_agg.at[plsc.Indices(sbuf_v, ignored_value=-1)]
    pltpu.make_async_copy(rows_v, dst, sem).wait()

  def batch_body(b, carry):
    pltpu.sync_copy(e0r_hbm.at[wid, b], e0t_v)
    pltpu.sync_copy(e1r_hbm.at[wid, b], e1t_v)
    pltpu.sync_copy(mr_hbm.at[wid, b], mt_v)
    issue_gather(0, rows_a, semg_a)

    def pair_body(i, carry2):
      c0 = 2 * i
      c1 = 2 * i + 1
      wait_gather(c0, rows_a, semg_a)

      @pl.when(i > 0)
      def _drain_b():
        wait_scatter(rows_b, sbuf_b, sems_b)

      issue_gather(c1, rows_b, semg_b)
      keeps = compute_scale(c0, rows_a)
      emit_outputs(c0, keeps, sbuf_a)
      issue_scatter(rows_a, sbuf_a, sems_a)
      wait_gather(c1, rows_b, semg_b)
      wait_scatter(rows_a, sbuf_a, sems_a)
      issue_gather(c0 + 2, rows_a, semg_a)
      keeps = compute_scale(c1, rows_b)
      emit_outputs(c1, keeps, sbuf_b)
      issue_scatter(rows_b, sbuf_b, sems_b)
      return carry2

    lax.fori_loop(0, BCH // 2, pair_body, 0)
    ct = BCH - 1
    wait_gather(ct, rows_a, semg_a)
    wait_scatter(rows_b, sbuf_b, sems_b)
    keeps = compute_scale(ct, rows_a)
    emit_outputs(ct, keeps, sbuf_a)
    pl.delay(200)
    pltpu.sync_copy(rows_a, sh_agg.at[plsc.Indices(sbuf_a, ignored_value=-1)],
                    add=True)
    pltpu.sync_copy(mt_v, pk_out.at[wid, b])
    return carry

  lax.fori_loop(0, NB, batch_body, 0)
  plsc.subcore_barrier()

  @pl.when(t < 10)
  def _export():
    pltpu.sync_copy(sh_agg.at[pl.ds(1000 * t, 1000)],
                    part_out.at[pl.ds(cid * N + 1000 * t, 1000)])


def _comb_body(p_ref, emb_ref, rel_ref, h_ref, t_ref, r_ref, agg_ref, sc_ref):
  agg_ref[...] = p_ref[0] + p_ref[1]
  h = h_ref[0]
  tt = t_ref[0]
  rr = r_ref[0]
  hrow = emb_ref[pl.ds(h, 1), :]
  trow = emb_ref[pl.ds(tt, 1), :]
  rrow = rel_ref[pl.ds(rr, 1), :]
  sc_ref[0, 0] = jnp.sum(hrow * rrow * trow)


_bfs_call = pl.kernel(
    _bfs_body,
    out_type=(
        jax.ShapeDtypeStruct((NR, 128), _i32),
    ),
    mesh=plsc.VectorSubcoreMesh(core_axis_name="c", subcore_axis_name="s"),
    compiler_params=pltpu.CompilerParams(needs_layout_passes=False),
    scratch_types=(
        pltpu.VMEM((NR, 128), _i32),
        pltpu.VMEM((NR, 128), _i32),
        pltpu.VMEM((NR, 128), _i32),
        pltpu.VMEM((NR, 128), _i32),
        pltpu.VMEM((EA,), _i32),
        pltpu.VMEM((EA,), _i32),
        pltpu.VMEM((16,), _i32),
        pltpu.VMEM((8, 128), _i32),
        pltpu.VMEM((NR,), _i32),
        pltpu.VMEM_SHARED((NR, 128), _i32),
    ),
)

_msg_call = pl.kernel(
    _msg_body,
    out_type=(
        jax.ShapeDtypeStruct((NC * NS, NB, BCH, CB), _i32),
        jax.ShapeDtypeStruct((NC * N, D), _f32),
    ),
    mesh=plsc.VectorSubcoreMesh(core_axis_name="c", subcore_axis_name="s"),
    compiler_params=pltpu.CompilerParams(needs_layout_passes=False),
    scratch_types=(
        pltpu.VMEM((NR, 128), _i32),
        pltpu.VMEM((BCH, CB), _i32),
        pltpu.VMEM((BCH, CB), _i32),
        pltpu.VMEM((BCH, CB), _i32),
        pltpu.VMEM((CB,), _i32),
        pltpu.VMEM((CB,), _i32),
        pltpu.VMEM((CB, D), _f32),
        pltpu.VMEM((CB, D), _f32),
        pltpu.VMEM_SHARED((N, D), _f32),
        pltpu.SemaphoreType.DMA,
        pltpu.SemaphoreType.DMA,
        pltpu.SemaphoreType.DMA,
        pltpu.SemaphoreType.DMA,
    ),
)

_comb_call = pl.pallas_call(
    _comb_body,
    out_shape=(
        jax.ShapeDtypeStruct((N, D), _f32),
        jax.ShapeDtypeStruct((1, 1), _f32),
    ),
    in_specs=[
        pl.BlockSpec(memory_space=pltpu.VMEM),
        pl.BlockSpec(memory_space=pltpu.VMEM),
        pl.BlockSpec(memory_space=pltpu.VMEM),
        pl.BlockSpec(memory_space=pltpu.SMEM),
        pl.BlockSpec(memory_space=pltpu.SMEM),
        pl.BlockSpec(memory_space=pltpu.SMEM),
    ],
    out_specs=(
        pl.BlockSpec(memory_space=pltpu.VMEM),
        pl.BlockSpec(memory_space=pltpu.SMEM),
    ),
)


def kernel(node_emb, rel_emb, edge_mask, edge_index, edge_type,
           head_idx, tail_idx, rel_idx):
  del edge_type
  e0 = edge_index[0]
  e1 = edge_index[1]
  h = jnp.asarray(head_idx, _i32)
  t = jnp.asarray(tail_idx, _i32)
  r = jnp.asarray(rel_idx, _i32)
  init_idx = jnp.concatenate(
      [h[None], t[None], jnp.broadcast_to(h[None], (14,))])
  zero_i = jnp.zeros((NR, 128), _i32)
  zero_f = jnp.zeros((1000, D), _f32)

  (enc2d,) = _bfs_call(e0, e1, init_idx, zero_i)
  e0r = e0.reshape(NC * NS, NB, BCH, CB)
  e1r = e1.reshape(NC * NS, NB, BCH, CB)
  mr = lax.bitcast_convert_type(edge_mask, _i32).reshape(NC * NS, NB, BCH, CB)
  pk, parts = _msg_call(node_emb, e0r, e1r, mr, enc2d, zero_f)
  p = parts.reshape(NC, N, D)
  agg, score = _comb_call(p, node_emb, rel_emb, h[None], t[None], r[None])
  pk = pk.reshape(E)
  sub_src = (pk & 0x3FFF) - 1
  sub_dst = ((pk >> 14) & 0x3FFF) - 1
  keep = (pk >> 28).astype(bool)
  return (agg, score.reshape(()), sub_src, sub_dst, keep)


# speedup vs baseline: 1.0613x; 1.0613x over previous
import functools

import jax
import jax.numpy as jnp
from jax import lax
from jax.experimental import pallas as pl
from jax.experimental.pallas import tpu as pltpu
from jax.experimental.pallas import tpu_sc as plsc

N = 10000
E = 320000
D = 128
NPAD = 10240
NR = NPAD // 128
NC = 2
NS = 16
EA = E // NS
CA = 2000
EW = E // (NC * NS)
CB = 80
NCH = EW // CB
BCH = 25
NB = NCH // BCH

_i32 = jnp.int32
_f32 = jnp.float32


def _rowcol(idx):
  return [lax.shift_right_logical(idx, 7), lax.bitwise_and(idx, 127)]


def _bfs_body(e0_hbm, e1_hbm, init_hbm, zero_hbm,
              enc_out,
              cur_v, all_v, new_v, tmp_v, e0c_v, e1c_v, init_v,
              stage_v, idvec_v, sh_hits):
  t = lax.axis_index("s")
  ones = jnp.ones((16,), _i32)
  lanes = jnp.arange(16, dtype=_i32)

  pltpu.sync_copy(zero_hbm, cur_v)
  pltpu.sync_copy(zero_hbm, all_v)
  pltpu.sync_copy(init_hbm, init_v)
  iv = init_v[...]
  plsc.store_scatter(cur_v, _rowcol(iv), ones)
  plsc.store_scatter(all_v, _rowcol(iv), ones)

  for g in range(NR // 16):
    idvec_v[pl.ds(16 * g, 16)] = lanes + 16 * g

  pltpu.sync_copy(e0_hbm.at[pl.ds(t * EA, EA)], e0c_v)
  pltpu.sync_copy(e1_hbm.at[pl.ds(t * EA, EA)], e1c_v)

  for _hop in range(2):
    pltpu.sync_copy(zero_hbm, new_v)

    @pl.when(t < 10)
    def _zero_hits():
      pltpu.sync_copy(zero_hbm.at[pl.ds(8 * t, 8)], sh_hits.at[pl.ds(8 * t, 8)])

    plsc.subcore_barrier()

    def chunk_body(ci, carry):
      for g in range(5):
        gi = 80 * ci + 16 * g
        a = e0c_v[pl.ds(gi, 16)]
        b = e1c_v[pl.ds(gi, 16)]
        sm = plsc.load_gather(cur_v, _rowcol(a))
        dm = plsc.load_gather(cur_v, _rowcol(b))
        plsc.store_scatter(new_v, _rowcol(b), ones, mask=sm > 0)
        plsc.store_scatter(new_v, _rowcol(a), ones, mask=dm > 0)
      return carry

    lax.fori_loop(0, EA // 80, chunk_body, 0)
    pl.delay(200)
    pltpu.sync_copy(new_v, sh_hits.at[idvec_v], add=True)
    plsc.subcore_barrier()
    pltpu.sync_copy(sh_hits, tmp_v)

    def rb_body(i, carry):
      for j in range(8):
        h = tmp_v[i, pl.ds(16 * j, 16)]
        c = (h > 0).astype(_i32)
        cur_v[i, pl.ds(16 * j, 16)] = c
        all_v[i, pl.ds(16 * j, 16)] = lax.bitwise_or(all_v[i, pl.ds(16 * j, 16)], c)
      return carry

    lax.fori_loop(0, NR, rb_body, 0)
    plsc.subcore_barrier()

  def pre_body(i, acc):
    for j in range(8):
      acc = acc + all_v[i, pl.ds(16 * j, 16)]
    return acc

  acc0 = lax.fori_loop(0, 8 * t, pre_body, jnp.zeros((16,), _i32))
  run0 = jnp.full((16,), jnp.sum(acc0), _i32)

  def cs_body(i_loc, run):
    for j in range(8):
      x = all_v[8 * t + i_loc, pl.ds(16 * j, 16)]
      pre = plsc.cumsum(x) + run
      stage_v[i_loc, pl.ds(16 * j, 16)] = lax.bitwise_or(
          lax.shift_left(pre, 1), x)
      run = run + jnp.full((16,), jnp.sum(x), _i32)
    return run

  nrows = jnp.where(t < 10, 8, 0)
  lax.fori_loop(0, nrows, cs_body, run0)
  pl.delay(200)

  @pl.when(t < 10)
  def _writeback():
    pltpu.sync_copy(stage_v, enc_out.at[pl.ds(8 * t, 8)])


def _msg_body(node_hbm, e0r_hbm, e1r_hbm, mr_hbm, enc_hbm, zf_hbm,
              pk_out, part_out,
              enc_v, e0t_v, e1t_v, mt_v, sbuf_a, sbuf_b,
              rows_a, rows_b, sh_agg, semg_a, semg_b, sems_a, sems_b):
  cid = lax.axis_index("c")
  t = lax.axis_index("s")
  wid = cid * NS + t
  neg1 = jnp.full((16,), -1, _i32)
  ones = jnp.ones((16,), _i32)
  lanes = jnp.arange(16, dtype=_i32)
  zf = jnp.zeros((16,), _f32)

  pltpu.sync_copy(enc_hbm, enc_v)

  @pl.when(t < 10)
  def _zero_agg():
    pltpu.sync_copy(zf_hbm, sh_agg.at[pl.ds(1000 * t, 1000)])

  plsc.subcore_barrier()

  def issue_gather(ci, rows_v, sem):
    return pltpu.async_copy(node_hbm.at[e0t_v.at[ci]], rows_v, sem)

  def wait_gather(ci, rows_v, sem):
    pltpu.make_async_copy(node_hbm.at[e0t_v.at[ci]], rows_v, sem).wait()

  def compute_scale(ci, rows_v):
    keeps = []
    for g in range(CB // 16):
      a = e0t_v[ci, pl.ds(16 * g, 16)]
      b = e1t_v[ci, pl.ds(16 * g, 16)]
      m = plsc.bitcast(mt_v[ci, pl.ds(16 * g, 16)], _f32)
      enc0 = plsc.load_gather(enc_v, _rowcol(a))
      enc1 = plsc.load_gather(enc_v, _rowcol(b))
      keep = lax.bitwise_and(lax.bitwise_and(enc0, enc1), ones)
      keeps.append((enc0, enc1, keep))
      w = keep.astype(_f32) / (1.0 + jnp.exp(-m))

      @plsc.parallel_loop(0, 16, 1, unroll=2)
      def _scale(r):
        ws = jnp.take_along_axis(w, jnp.full((16,), r, _i32), axis=0)
        for k in range(8):
          rows_v[16 * g + r, pl.ds(16 * k, 16)] = (
              rows_v[16 * g + r, pl.ds(16 * k, 16)] * ws)

    return keeps

  def emit_outputs(ci, keeps, sbuf_v):
    kbit = jnp.full((16,), 1 << 28, _i32)
    zi = jnp.zeros((16,), _i32)
    for g in range(CB // 16):
      enc0, enc1, keep = keeps[g]
      b = e1t_v[ci, pl.ds(16 * g, 16)]
      kb = keep > 0
      sbuf_v[pl.ds(16 * g, 16)] = jnp.where(kb, b, neg1)
      hi = lax.bitwise_or(
          lax.bitwise_or(lax.shift_right_logical(enc0, 1),
                         lax.shift_left(lax.shift_right_logical(enc1, 1), 14)),
          kbit)
      mt_v[ci, pl.ds(16 * g, 16)] = jnp.where(kb, hi, zi)

  def issue_scatter(rows_v, sbuf_v, sem):
    pl.delay(100)
    dst = sh_agg.at[plsc.Indices(sbuf_v, ignored_value=-1)]
    return pltpu.async_copy(rows_v, dst, sem, add=True)

  def wait_scatter(rows_v, sbuf_v, sem):
    dst = sh_agg.at[plsc.Indices(sbuf_v, ignored_value=-1)]
    pltpu.make_async_copy(rows_v, dst, sem).wait()

  def batch_body(b, carry):
    pltpu.sync_copy(e0r_hbm.at[wid, b], e0t_v)
    pltpu.sync_copy(e1r_hbm.at[wid, b], e1t_v)
    pltpu.sync_copy(mr_hbm.at[wid, b], mt_v)
    issue_gather(0, rows_a, semg_a)

    def pair_body(i, carry2):
      c0 = 2 * i
      c1 = 2 * i + 1
      wait_gather(c0, rows_a, semg_a)

      @pl.when(i > 0)
      def _drain_b():
        wait_scatter(rows_b, sbuf_b, sems_b)

      issue_gather(c1, rows_b, semg_b)
      keeps = compute_scale(c0, rows_a)
      emit_outputs(c0, keeps, sbuf_a)
      issue_scatter(rows_a, sbuf_a, sems_a)
      wait_gather(c1, rows_b, semg_b)
      keeps = compute_scale(c1, rows_b)
      emit_outputs(c1, keeps, sbuf_b)
      wait_scatter(rows_a, sbuf_a, sems_a)
      issue_gather(c0 + 2, rows_a, semg_a)
      issue_scatter(rows_b, sbuf_b, sems_b)
      return carry2

    lax.fori_loop(0, BCH // 2, pair_body, 0)
    ct = BCH - 1
    wait_gather(ct, rows_a, semg_a)
    wait_scatter(rows_b, sbuf_b, sems_b)
    keeps = compute_scale(ct, rows_a)
    emit_outputs(ct, keeps, sbuf_a)
    pl.delay(200)
    pltpu.sync_copy(rows_a, sh_agg.at[plsc.Indices(sbuf_a, ignored_value=-1)],
                    add=True)
    pltpu.sync_copy(mt_v, pk_out.at[wid, b])
    return carry

  lax.fori_loop(0, NB, batch_body, 0)
  plsc.subcore_barrier()

  @pl.when(t < 10)
  def _export():
    pltpu.sync_copy(sh_agg.at[pl.ds(1000 * t, 1000)],
                    part_out.at[pl.ds(cid * N + 1000 * t, 1000)])


def _comb_body(p_ref, emb_ref, rel_ref, h_ref, t_ref, r_ref, agg_ref, sc_ref):
  agg_ref[...] = p_ref[0] + p_ref[1]
  h = h_ref[0]
  tt = t_ref[0]
  rr = r_ref[0]
  hrow = emb_ref[pl.ds(h, 1), :]
  trow = emb_ref[pl.ds(tt, 1), :]
  rrow = rel_ref[pl.ds(rr, 1), :]
  sc_ref[0, 0] = jnp.sum(hrow * rrow * trow)


_bfs_call = pl.kernel(
    _bfs_body,
    out_type=(
        jax.ShapeDtypeStruct((NR, 128), _i32),
    ),
    mesh=plsc.VectorSubcoreMesh(core_axis_name="c", subcore_axis_name="s"),
    compiler_params=pltpu.CompilerParams(needs_layout_passes=False),
    scratch_types=(
        pltpu.VMEM((NR, 128), _i32),
        pltpu.VMEM((NR, 128), _i32),
        pltpu.VMEM((NR, 128), _i32),
        pltpu.VMEM((NR, 128), _i32),
        pltpu.VMEM((EA,), _i32),
        pltpu.VMEM((EA,), _i32),
        pltpu.VMEM((16,), _i32),
        pltpu.VMEM((8, 128), _i32),
        pltpu.VMEM((NR,), _i32),
        pltpu.VMEM_SHARED((NR, 128), _i32),
    ),
)

_msg_call = pl.kernel(
    _msg_body,
    out_type=(
        jax.ShapeDtypeStruct((NC * NS, NB, BCH, CB), _i32),
        jax.ShapeDtypeStruct((NC * N, D), _f32),
    ),
    mesh=plsc.VectorSubcoreMesh(core_axis_name="c", subcore_axis_name="s"),
    compiler_params=pltpu.CompilerParams(needs_layout_passes=False),
    scratch_types=(
        pltpu.VMEM((NR, 128), _i32),
        pltpu.VMEM((BCH, CB), _i32),
        pltpu.VMEM((BCH, CB), _i32),
        pltpu.VMEM((BCH, CB), _i32),
        pltpu.VMEM((CB,), _i32),
        pltpu.VMEM((CB,), _i32),
        pltpu.VMEM((CB, D), _f32),
        pltpu.VMEM((CB, D), _f32),
        pltpu.VMEM_SHARED((N, D), _f32),
        pltpu.SemaphoreType.DMA,
        pltpu.SemaphoreType.DMA,
        pltpu.SemaphoreType.DMA,
        pltpu.SemaphoreType.DMA,
    ),
)

_comb_call = pl.pallas_call(
    _comb_body,
    out_shape=(
        jax.ShapeDtypeStruct((N, D), _f32),
        jax.ShapeDtypeStruct((1, 1), _f32),
    ),
    in_specs=[
        pl.BlockSpec(memory_space=pltpu.VMEM),
        pl.BlockSpec(memory_space=pltpu.VMEM),
        pl.BlockSpec(memory_space=pltpu.VMEM),
        pl.BlockSpec(memory_space=pltpu.SMEM),
        pl.BlockSpec(memory_space=pltpu.SMEM),
        pl.BlockSpec(memory_space=pltpu.SMEM),
    ],
    out_specs=(
        pl.BlockSpec(memory_space=pltpu.VMEM),
        pl.BlockSpec(memory_space=pltpu.SMEM),
    ),
)


def kernel(node_emb, rel_emb, edge_mask, edge_index, edge_type,
           head_idx, tail_idx, rel_idx):
  del edge_type
  e0 = edge_index[0]
  e1 = edge_index[1]
  h = jnp.asarray(head_idx, _i32)
  t = jnp.asarray(tail_idx, _i32)
  r = jnp.asarray(rel_idx, _i32)
  init_idx = jnp.concatenate(
      [h[None], t[None], jnp.broadcast_to(h[None], (14,))])
  zero_i = jnp.zeros((NR, 128), _i32)
  zero_f = jnp.zeros((1000, D), _f32)

  (enc2d,) = _bfs_call(e0, e1, init_idx, zero_i)
  e0r = e0.reshape(NC * NS, NB, BCH, CB)
  e1r = e1.reshape(NC * NS, NB, BCH, CB)
  mr = lax.bitcast_convert_type(edge_mask, _i32).reshape(NC * NS, NB, BCH, CB)
  pk, parts = _msg_call(node_emb, e0r, e1r, mr, enc2d, zero_f)
  p = parts.reshape(NC, N, D)
  agg, score = _comb_call(p, node_emb, rel_emb, h[None], t[None], r[None])
  pk = pk.reshape(E)
  sub_src = (pk & 0x3FFF) - 1
  sub_dst = ((pk >> 14) & 0x3FFF) - 1
  keep = (pk >> 28).astype(bool)
  return (agg, score.reshape(()), sub_src, sub_dst, keep)


# speedup vs baseline: 1.1372x; 1.0715x over previous
import functools

import jax
import jax.numpy as jnp
from jax import lax
from jax.experimental import pallas as pl
from jax.experimental.pallas import tpu as pltpu
from jax.experimental.pallas import tpu_sc as plsc

N = 10000
E = 320000
D = 128
NPAD = 10240
NR = NPAD // 128
NC = 2
NS = 16
EA = E // NS
CA = 2000
EW = E // (NC * NS)
CB = 80
NCH = EW // CB
BCH = 25
NB = NCH // BCH

_i32 = jnp.int32
_f32 = jnp.float32


def _rowcol(idx):
  return [lax.shift_right_logical(idx, 7), lax.bitwise_and(idx, 127)]


def _bfs_body(e0_hbm, e1_hbm, m_hbm, init_hbm, zero_hbm,
              pk_out, gm_out, sm_out, w_out,
              cur_v, all_v, new_v, e0c_v, e1c_v, mall_v, init_v,
              stage_v, idvec_v, pkq_v, g0q_v, g1q_v, wq_v, sh_hits):
  t = lax.axis_index("s")
  ones = jnp.ones((16,), _i32)
  lanes = jnp.arange(16, dtype=_i32)

  pltpu.sync_copy(zero_hbm, cur_v)
  pltpu.sync_copy(zero_hbm, all_v)
  pltpu.sync_copy(init_hbm, init_v)
  iv = init_v[...]
  plsc.store_scatter(cur_v, _rowcol(iv), ones)
  plsc.store_scatter(all_v, _rowcol(iv), ones)

  for g in range(NR // 16):
    idvec_v[pl.ds(16 * g, 16)] = lanes + 16 * g

  pltpu.sync_copy(e0_hbm.at[pl.ds(t * EA, EA)], e0c_v)
  pltpu.sync_copy(e1_hbm.at[pl.ds(t * EA, EA)], e1c_v)
  pltpu.sync_copy(m_hbm.at[pl.ds(t * EA, EA)], mall_v)

  for _hop in range(2):
    pltpu.sync_copy(zero_hbm, new_v)

    @pl.when(t < 10)
    def _zero_hits():
      pltpu.sync_copy(zero_hbm.at[pl.ds(8 * t, 8)], sh_hits.at[pl.ds(8 * t, 8)])

    plsc.subcore_barrier()

    def chunk_body(ci, carry):
      for g in range(5):
        gi = 80 * ci + 16 * g
        a = e0c_v[pl.ds(gi, 16)]
        b = e1c_v[pl.ds(gi, 16)]
        sm = plsc.load_gather(cur_v, _rowcol(a))
        dm = plsc.load_gather(cur_v, _rowcol(b))
        plsc.store_scatter(new_v, _rowcol(b), ones, mask=sm > 0)
        plsc.store_scatter(new_v, _rowcol(a), ones, mask=dm > 0)
      return carry

    lax.fori_loop(0, EA // 80, chunk_body, 0)
    pl.delay(200)
    pltpu.sync_copy(new_v, sh_hits.at[idvec_v], add=True)
    plsc.subcore_barrier()
    pltpu.sync_copy(sh_hits, new_v)

    def rb_body(i, carry):
      for j in range(8):
        h = new_v[i, pl.ds(16 * j, 16)]
        c = (h > 0).astype(_i32)
        cur_v[i, pl.ds(16 * j, 16)] = c
        all_v[i, pl.ds(16 * j, 16)] = lax.bitwise_or(all_v[i, pl.ds(16 * j, 16)], c)
      return carry

    lax.fori_loop(0, NR, rb_body, 0)
    plsc.subcore_barrier()

  def pre_body(i, acc):
    for j in range(8):
      acc = acc + all_v[i, pl.ds(16 * j, 16)]
    return acc

  acc0 = lax.fori_loop(0, 8 * t, pre_body, jnp.zeros((16,), _i32))
  run0 = jnp.full((16,), jnp.sum(acc0), _i32)

  def cs_body(i_loc, run):
    for j in range(8):
      x = all_v[8 * t + i_loc, pl.ds(16 * j, 16)]
      pre = plsc.cumsum(x) + run
      stage_v[i_loc, pl.ds(16 * j, 16)] = lax.bitwise_or(
          lax.shift_left(pre, 1), x)
      run = run + jnp.full((16,), jnp.sum(x), _i32)
    return run

  nrows = jnp.where(t < 10, 8, 0)
  lax.fori_loop(0, nrows, cs_body, run0)
  pl.delay(200)

  @pl.when(t < 10)
  def _writeback():
    pltpu.sync_copy(stage_v, sh_hits.at[pl.ds(8 * t, 8)])

  plsc.subcore_barrier()
  pltpu.sync_copy(sh_hits, cur_v)

  kbit = jnp.full((16,), 1 << 28, _i32)
  zi = jnp.zeros((16,), _i32)
  neg1 = jnp.full((16,), -1, _i32)
  for q in range(5):
    def eg_body(i, carry):
      gi = (q * 250 + i) * 16
      a = e0c_v[pl.ds(gi, 16)]
      b = e1c_v[pl.ds(gi, 16)]
      m = mall_v[pl.ds(gi, 16)]
      enc0 = plsc.load_gather(cur_v, _rowcol(a))
      enc1 = plsc.load_gather(cur_v, _rowcol(b))
      kb = lax.bitwise_and(lax.bitwise_and(enc0, enc1), ones) > 0
      hi = lax.bitwise_or(
          lax.bitwise_or(lax.shift_right_logical(enc0, 1),
                         lax.shift_left(lax.shift_right_logical(enc1, 1), 14)),
          kbit)
      o = 16 * i
      pkq_v[pl.ds(o, 16)] = jnp.where(kb, hi, zi)
      g0q_v[pl.ds(o, 16)] = jnp.where(kb, a, neg1)
      g1q_v[pl.ds(o, 16)] = jnp.where(kb, b, neg1)
      wq_v[pl.ds(o, 16)] = 1.0 / (1.0 + jnp.exp(-m))
      return carry

    lax.fori_loop(0, 250, eg_body, 0)
    pl.delay(200)
    base = t * EA + q * 4000
    pltpu.sync_copy(pkq_v, pk_out.at[pl.ds(base, 4000)])
    pltpu.sync_copy(g0q_v, gm_out.at[pl.ds(base, 4000)])
    pltpu.sync_copy(g1q_v, sm_out.at[pl.ds(base, 4000)])
    pltpu.sync_copy(wq_v, w_out.at[pl.ds(base, 4000)])


def _msg_body(node_hbm, gmr_hbm, smr_hbm, wr_hbm, zf_hbm,
              part_out,
              gt_v, st_v, wt_v, rows_a, rows_b, rows_c, sh_agg,
              semg_a, semg_b, semg_c, sems_a, sems_b, sems_c):
  cid = lax.axis_index("c")
  t = lax.axis_index("s")
  wid = cid * NS + t
  bufs = (rows_a, rows_b, rows_c)
  gsems = (semg_a, semg_b, semg_c)
  ssems = (sems_a, sems_b, sems_c)

  @pl.when(t < 10)
  def _zero_agg():
    pltpu.sync_copy(zf_hbm, sh_agg.at[pl.ds(1000 * t, 1000)])

  plsc.subcore_barrier()

  def issue_gather(ci, k):
    src = node_hbm.at[plsc.Indices(gt_v.at[ci], ignored_value=-1)]
    return pltpu.async_copy(src, bufs[k], gsems[k])

  def wait_gather(ci, k):
    src = node_hbm.at[plsc.Indices(gt_v.at[ci], ignored_value=-1)]
    pltpu.make_async_copy(src, bufs[k], gsems[k]).wait()

  def issue_scatter(ci, k):
    pl.delay(100)
    dst = sh_agg.at[plsc.Indices(st_v.at[ci], ignored_value=-1)]
    return pltpu.async_copy(bufs[k], dst, ssems[k], add=True)

  def wait_scatter(ci, k):
    dst = sh_agg.at[plsc.Indices(st_v.at[ci], ignored_value=-1)]
    pltpu.make_async_copy(bufs[k], dst, ssems[k]).wait()

  def scale(ci, k):
    rows_v = bufs[k]
    for g in range(CB // 16):
      w = wt_v[ci, pl.ds(16 * g, 16)]

      @plsc.parallel_loop(0, 16, 1, unroll=2)
      def _scale(r):
        ws = jnp.take_along_axis(w, jnp.full((16,), r, _i32), axis=0)
        for kk in range(8):
          rows_v[16 * g + r, pl.ds(16 * kk, 16)] = (
              rows_v[16 * g + r, pl.ds(16 * kk, 16)] * ws)

  def batch_body(b, carry):
    pltpu.sync_copy(gmr_hbm.at[wid, b], gt_v)
    pltpu.sync_copy(smr_hbm.at[wid, b], st_v)
    pltpu.sync_copy(wr_hbm.at[wid, b], wt_v)
    issue_gather(0, 0)
    issue_gather(1, 1)

    def step(c, k):
      wait_gather(c, k)
      scale(c, k)
      issue_scatter(c, k)
      kprev = (k + 2) % 3

      @pl.when(c > 0)
      def _drain():
        wait_scatter(c - 1, kprev)

      @pl.when(c + 2 < BCH)
      def _prefetch():
        issue_gather(c + 2, kprev)

    def triple_body(i, carry2):
      for k in range(3):
        step(3 * i + k, k)
      return carry2

    lax.fori_loop(0, BCH // 3, triple_body, 0)
    ct = BCH - 1
    step(ct, ct % 3)
    wait_scatter(ct, ct % 3)
    return carry

  lax.fori_loop(0, NB, batch_body, 0)
  plsc.subcore_barrier()

  @pl.when(t < 10)
  def _export():
    pltpu.sync_copy(sh_agg.at[pl.ds(1000 * t, 1000)],
                    part_out.at[pl.ds(cid * N + 1000 * t, 1000)])


def _comb_body(p_ref, emb_ref, rel_ref, h_ref, t_ref, r_ref, agg_ref, sc_ref):
  agg_ref[...] = p_ref[0] + p_ref[1]
  h = h_ref[0]
  tt = t_ref[0]
  rr = r_ref[0]
  hrow = emb_ref[pl.ds(h, 1), :]
  trow = emb_ref[pl.ds(tt, 1), :]
  rrow = rel_ref[pl.ds(rr, 1), :]
  sc_ref[0, 0] = jnp.sum(hrow * rrow * trow)


_bfs_call = pl.kernel(
    _bfs_body,
    out_type=(
        jax.ShapeDtypeStruct((E,), _i32),
        jax.ShapeDtypeStruct((E,), _i32),
        jax.ShapeDtypeStruct((E,), _i32),
        jax.ShapeDtypeStruct((E,), _f32),
    ),
    mesh=plsc.VectorSubcoreMesh(core_axis_name="c", subcore_axis_name="s"),
    compiler_params=pltpu.CompilerParams(needs_layout_passes=False),
    scratch_types=(
        pltpu.VMEM((NR, 128), _i32),
        pltpu.VMEM((NR, 128), _i32),
        pltpu.VMEM((NR, 128), _i32),
        pltpu.VMEM((EA,), _i32),
        pltpu.VMEM((EA,), _i32),
        pltpu.VMEM((EA,), _f32),
        pltpu.VMEM((16,), _i32),
        pltpu.VMEM((8, 128), _i32),
        pltpu.VMEM((NR,), _i32),
        pltpu.VMEM((4000,), _i32),
        pltpu.VMEM((4000,), _i32),
        pltpu.VMEM((4000,), _i32),
        pltpu.VMEM((4000,), _f32),
        pltpu.VMEM_SHARED((NR, 128), _i32),
    ),
)

_msg_call = pl.kernel(
    _msg_body,
    out_type=(
        jax.ShapeDtypeStruct((NC * N, D), _f32),
    ),
    mesh=plsc.VectorSubcoreMesh(core_axis_name="c", subcore_axis_name="s"),
    compiler_params=pltpu.CompilerParams(needs_layout_passes=False),
    scratch_types=(
        pltpu.VMEM((BCH, CB), _i32),
        pltpu.VMEM((BCH, CB), _i32),
        pltpu.VMEM((BCH, CB), _f32),
        pltpu.VMEM((CB, D), _f32),
        pltpu.VMEM((CB, D), _f32),
        pltpu.VMEM((CB, D), _f32),
        pltpu.VMEM_SHARED((N, D), _f32),
        pltpu.SemaphoreType.DMA,
        pltpu.SemaphoreType.DMA,
        pltpu.SemaphoreType.DMA,
        pltpu.SemaphoreType.DMA,
        pltpu.SemaphoreType.DMA,
        pltpu.SemaphoreType.DMA,
    ),
)

_comb_call = pl.pallas_call(
    _comb_body,
    out_shape=(
        jax.ShapeDtypeStruct((N, D), _f32),
        jax.ShapeDtypeStruct((1, 1), _f32),
    ),
    in_specs=[
        pl.BlockSpec(memory_space=pltpu.VMEM),
        pl.BlockSpec(memory_space=pltpu.VMEM),
        pl.BlockSpec(memory_space=pltpu.VMEM),
        pl.BlockSpec(memory_space=pltpu.SMEM),
        pl.BlockSpec(memory_space=pltpu.SMEM),
        pl.BlockSpec(memory_space=pltpu.SMEM),
    ],
    out_specs=(
        pl.BlockSpec(memory_space=pltpu.VMEM),
        pl.BlockSpec(memory_space=pltpu.SMEM),
    ),
)


def kernel(node_emb, rel_emb, edge_mask, edge_index, edge_type,
           head_idx, tail_idx, rel_idx):
  del edge_type
  e0 = edge_index[0]
  e1 = edge_index[1]
  h = jnp.asarray(head_idx, _i32)
  t = jnp.asarray(tail_idx, _i32)
  r = jnp.asarray(rel_idx, _i32)
  init_idx = jnp.concatenate(
      [h[None], t[None], jnp.broadcast_to(h[None], (14,))])
  zero_i = jnp.zeros((NR, 128), _i32)
  zero_f = jnp.zeros((1000, D), _f32)

  pk, gm, sm, wv = _bfs_call(e0, e1, edge_mask, init_idx, zero_i)
  gmr = gm.reshape(NC * NS, NB, BCH, CB)
  smr = sm.reshape(NC * NS, NB, BCH, CB)
  wr = wv.reshape(NC * NS, NB, BCH, CB)
  (parts,) = _msg_call(node_emb, gmr, smr, wr, zero_f)
  p = parts.reshape(NC, N, D)
  agg, score = _comb_call(p, node_emb, rel_emb, h[None], t[None], r[None])
  sub_src = (pk & 0x3FFF) - 1
  sub_dst = ((pk >> 14) & 0x3FFF) - 1
  keep = (pk >> 28).astype(bool)
  return (agg, score.reshape(()), sub_src, sub_dst, keep)
